# baseline (speedup 1.0000x reference)
import functools

import jax
import jax.numpy as jnp
from jax import lax
from jax.experimental import pallas as pl
from jax.experimental.pallas import tpu as pltpu
from jax.experimental.pallas import tpu_sc as plsc

_B, _S, _D, _R = 4096, 50, 64, 16
_N = _B * _S
_NW = 32
_ROWS_PER_W = _N // _NW
_CH = 128
_NCH = _ROWS_PER_W // _CH
_L = 16


def _sc_embed_lora(x_flat, table2, lora_A2, lora_B):
    mesh = plsc.VectorSubcoreMesh(core_axis_name="c", subcore_axis_name="s")

    @functools.partial(
        pl.kernel,
        out_type=jax.ShapeDtypeStruct((_N, _D), jnp.float32),
        mesh=mesh,
        scratch_types=[
            pltpu.VMEM((_ROWS_PER_W,), jnp.int32),
            pltpu.VMEM((_ROWS_PER_W,), jnp.int32),
            pltpu.VMEM((_ROWS_PER_W,), jnp.int32),
            pltpu.VMEM((_CH, 128), jnp.float32),
            pltpu.VMEM((_CH, 128), jnp.float32),
            pltpu.VMEM((_CH, 128), jnp.float32),
            pltpu.VMEM((_CH, 128), jnp.float32),
            pltpu.VMEM((_CH, _D), jnp.float32),
            pltpu.VMEM((_CH, _D), jnp.float32),
            pltpu.VMEM((_R, _D), jnp.float32),
            pltpu.SemaphoreType.DMA,
            pltpu.SemaphoreType.DMA,
            pltpu.SemaphoreType.DMA,
            pltpu.SemaphoreType.DMA,
            pltpu.SemaphoreType.DMA,
            pltpu.SemaphoreType.DMA,
        ],
    )
    def k(x_hbm, tbl_hbm, a_hbm, b_hbm, out_hbm,
          idx_v, tix_v, aix_v, g0, g1, a0, a1, o0, o1, b_v,
          st0, st1, sa0, sa1, so0, so1):
        wid = lax.axis_index("s") * 2 + lax.axis_index("c")
        base = wid * _ROWS_PER_W
        pltpu.sync_copy(b_hbm, b_v)
        pltpu.sync_copy(x_hbm.at[pl.ds(base, _ROWS_PER_W)], idx_v)

        def shift_body(kk, c):
            v = idx_v[pl.ds(kk * _L, _L)]
            tix_v[pl.ds(kk * _L, _L)] = lax.shift_right_logical(v, 1)
            aix_v[pl.ds(kk * _L, _L)] = lax.shift_right_logical(v, 3)
            return c

        lax.fori_loop(0, _ROWS_PER_W // _L, shift_body, 0)

        gbuf = (g0, g1)
        abuf = (a0, a1)
        obuf = (o0, o1)
        sts = (st0, st1)
        sas = (sa0, sa1)
        sos = (so0, so1)

        def issue_gather(g, c):
            t_ref = tix_v.at[pl.ds(g * _CH, _CH)]
            a_ref = aix_v.at[pl.ds(g * _CH, _CH)]
            pltpu.async_copy(tbl_hbm.at[t_ref], gbuf[c], sts[c])
            pltpu.async_copy(a_hbm.at[a_ref], abuf[c], sas[c])

        def wait_gather(g, c):
            t_ref = tix_v.at[pl.ds(g * _CH, _CH)]
            a_ref = aix_v.at[pl.ds(g * _CH, _CH)]
            pltpu.make_async_copy(tbl_hbm.at[t_ref], gbuf[c], sts[c]).wait()
            pltpu.make_async_copy(a_hbm.at[a_ref], abuf[c], sas[c]).wait()

        def issue_store(g, c):
            pltpu.async_copy(obuf[c], out_hbm.at[pl.ds(base + g * _CH, _CH)],
                             sos[c])

        def wait_store(c):
            pltpu.make_async_copy(obuf[c],
                                  out_hbm.at[pl.ds(base, _CH)], sos[c]).wait()

        def compute_chunk(g, c):
            for p in range(2):
                bv = [(b_v[r, pl.ds(32 * p, 16)],
                       b_v[r, pl.ds(32 * p + 16, 16)]) for r in range(_R)]

                def blk_body(kk, cc, bv=bv, p=p, g=g, c=c):
                    ivec = idx_v[pl.ds(g * _CH + kk * _L, _L)]
                    for l in range(_L):
                        i = kk * _L + l
                        iv = ivec[l]
                        toff = lax.shift_left(
                            lax.bitwise_and(iv, 1), 6) + 32 * p
                        aoff = lax.shift_left(lax.bitwise_and(iv, 7), 4)
                        a_vec = abuf[c][i, pl.ds(aoff, 16)]
                        acc0 = gbuf[c][i, pl.ds(toff, 16)]
                        acc1 = gbuf[c][i, pl.ds(toff + 16, 16)]
                        for r in range(_R):
                            s = a_vec[r]
                            acc0 = acc0 + s * bv[r][0]
                            acc1 = acc1 + s * bv[r][1]
                        obuf[c][i, pl.ds(32 * p, 16)] = acc0
                        obuf[c][i, pl.ds(32 * p + 16, 16)] = acc1
                    return cc

                lax.fori_loop(0, _CH // _L, blk_body, 0)

        issue_gather(0, 0)
        issue_gather(1, 1)

        def body(t, carry):
            for c in range(2):
                g = 2 * t + c
                wait_gather(g, c)

                @pl.when(t > 0)
                def _():
                    wait_store(c)

                compute_chunk(g, c)
                issue_store(g, c)

                @pl.when(g + 2 < _NCH)
                def _():
                    issue_gather(g + 2, c)
            return carry

        lax.fori_loop(0, _NCH // 2, body, 0)
        wait_store(0)
        wait_store(1)

    return k(x_flat, table2, lora_A2, lora_B)


def kernel(x, table, lora_A, lora_B):
    x_flat = x.reshape(-1).astype(jnp.int32)
    table2 = table.reshape(500000, 128)
    lora_A2 = lora_A.reshape(125000, 128)
    out = _sc_embed_lora(x_flat, table2, lora_A2, lora_B)
    return out.reshape(_B, _S, _D)


# speedup vs baseline: 1.2224x; 1.2224x over previous
import functools

import jax
import jax.numpy as jnp
from jax import lax
from jax.experimental import pallas as pl
from jax.experimental.pallas import tpu as pltpu
from jax.experimental.pallas import tpu_sc as plsc

_B, _S, _D, _R = 4096, 50, 64, 16
_N = _B * _S
_NW = 32
_ROWS_PER_W = _N // _NW
_CH = 128
_NCH = _ROWS_PER_W // _CH
_L = 16


def _sc_embed_lora(x_flat, packed, lora_B):
    mesh = plsc.VectorSubcoreMesh(core_axis_name="c", subcore_axis_name="s")

    @functools.partial(
        pl.kernel,
        out_type=jax.ShapeDtypeStruct((_N, _D), jnp.float32),
        mesh=mesh,
        scratch_types=[
            pltpu.VMEM((_ROWS_PER_W,), jnp.int32),
            pltpu.VMEM((_CH, 128), jnp.float32),
            pltpu.VMEM((_CH, 128), jnp.float32),
            pltpu.VMEM((_CH, _D), jnp.float32),
            pltpu.VMEM((_CH, _D), jnp.float32),
            pltpu.VMEM((_R, _D), jnp.float32),
            pltpu.SemaphoreType.DMA,
            pltpu.SemaphoreType.DMA,
            pltpu.SemaphoreType.DMA,
            pltpu.SemaphoreType.DMA,
        ],
    )
    def k(x_hbm, pk_hbm, b_hbm, out_hbm,
          idx_v, p0, p1, o0, o1, b_v, sg0, sg1, so0, so1):
        wid = lax.axis_index("s") * 2 + lax.axis_index("c")
        base = wid * _ROWS_PER_W
        pltpu.sync_copy(b_hbm, b_v)
        pltpu.sync_copy(x_hbm.at[pl.ds(base, _ROWS_PER_W)], idx_v)

        pbuf = (p0, p1)
        obuf = (o0, o1)
        sgs = (sg0, sg1)
        sos = (so0, so1)

        def issue_gather(g, c):
            i_ref = idx_v.at[pl.ds(g * _CH, _CH)]
            pltpu.async_copy(pk_hbm.at[i_ref], pbuf[c], sgs[c])

        def wait_gather(g, c):
            i_ref = idx_v.at[pl.ds(g * _CH, _CH)]
            pltpu.make_async_copy(pk_hbm.at[i_ref], pbuf[c], sgs[c]).wait()

        def issue_store(g, c):
            pltpu.async_copy(obuf[c], out_hbm.at[pl.ds(base + g * _CH, _CH)],
                             sos[c])

        def wait_store(c):
            pltpu.make_async_copy(obuf[c],
                                  out_hbm.at[pl.ds(base, _CH)], sos[c]).wait()

        def compute_chunk(c):
            for p in range(2):
                bv = [(b_v[r, pl.ds(32 * p, 16)],
                       b_v[r, pl.ds(32 * p + 16, 16)]) for r in range(_R)]

                def row_body(i, cc, bv=bv, p=p):
                    a_vec = pbuf[c][i, pl.ds(64, 16)]
                    acc0 = pbuf[c][i, pl.ds(32 * p, 16)]
                    acc1 = pbuf[c][i, pl.ds(32 * p + 16, 16)]
                    for r in range(_R):
                        s = a_vec[r]
                        acc0 = acc0 + s * bv[r][0]
                        acc1 = acc1 + s * bv[r][1]
                    obuf[c][i, pl.ds(32 * p, 16)] = acc0
                    obuf[c][i, pl.ds(32 * p + 16, 16)] = acc1
                    return cc

                lax.fori_loop(0, _CH, row_body, 0)

        issue_gather(0, 0)
        issue_gather(1, 1)

        def body(t, carry):
            for c in range(2):
                g = 2 * t + c
                wait_gather(g, c)

                @pl.when(t > 0)
                def _():
                    wait_store(c)

                compute_chunk(c)
                issue_store(g, c)

                @pl.when(g + 2 < _NCH)
                def _():
                    issue_gather(g + 2, c)
            return carry

        lax.fori_loop(0, _NCH // 2, body, 0)
        wait_store(0)
        wait_store(1)

    return k(x_flat, packed, lora_B)


def kernel(x, table, lora_A, lora_B):
    x_flat = x.reshape(-1).astype(jnp.int32)
    zeros = jnp.zeros((table.shape[0], 128 - _D - _R), jnp.float32)
    packed = jnp.concatenate([table, lora_A, zeros], axis=1)
    out = _sc_embed_lora(x_flat, packed, lora_B)
    return out.reshape(_B, _S, _D)
